# baseline (speedup 1.0000x reference)
import math
import functools


import jax
import jax.numpy as jnp
from jax import lax
from jax.experimental import pallas as pl
from jax.experimental.pallas import tpu as pltpu

HIGHEST = jax.lax.Precision.HIGHEST
BF = jnp.bfloat16


def _bdot(a, b, dims=None):
    if dims is None:
        dims = (((a.ndim - 1,), (0,)), ((), ()))
    return lax.dot_general(a.astype(BF), b.astype(BF), dims,
                           preferred_element_type=jnp.float32)

B, T, D = 2, 4096, 1024
H, E = 16, 64
BH = B * H
NB = 128
BUCKET = 64
N_HASHES = 2
SOFTMAX_EPS = 1e-06
SM_TEMP = 1.0 / math.sqrt(E)
SQRT_TEMP = math.sqrt(SM_TEMP)
HALF_LOG_NB = 0.5 * math.log(NB)


def _prep_body(q_ref, k_ref, v_ref, alpha_ref, beta_ref, proj_ref,
               hq_ref, hk_ref, pls_ref, kstab_ref, qk1_ref, qkv_ref):
    q = q_ref[0]
    k = k_ref[0]
    v = v_ref[0]
    alpha = alpha_ref[...]
    beta = beta_ref[...]
    proj = proj_ref[...]

    qn2 = jnp.sum(q * q, axis=-1, keepdims=True)
    kn2 = jnp.sum(k * k, axis=-1, keepdims=True)
    qn = jnp.sqrt(qn2)
    kn = jnp.sqrt(kn2)
    mq = jnp.max(qn)
    mk = jnp.max(kn)
    q_extra = jnp.sqrt(jnp.maximum(mq * mq - qn * qn, 0.0))
    k_extra = jnp.sqrt(jnp.maximum(mk * mk - kn * kn, 0.0))
    zcol = jnp.zeros_like(qn)
    q_ext = jnp.concatenate([q, q_extra, zcol], axis=-1).astype(jnp.bfloat16)
    k_ext = jnp.concatenate([k, zcol, k_extra], axis=-1).astype(jnp.bfloat16)
    alpha_bf = alpha.astype(jnp.bfloat16)
    hq = jnp.dot(q_ext, alpha_bf, preferred_element_type=jnp.float32) + beta
    hk = jnp.dot(k_ext, alpha_bf, preferred_element_type=jnp.float32) + beta
    hq_ref[0] = hq
    hk_ref[0] = hk

    q_sc = _bdot(SQRT_TEMP * q, proj) - qn2 * (SM_TEMP * 0.5)
    k_sc = _bdot(SQRT_TEMP * k, proj) - kn2 * (SM_TEMP * 0.5)
    q_stab = jnp.max(q_sc, axis=-1, keepdims=True)
    k_stab = jnp.max(k_sc)
    q_prime = jnp.exp(q_sc - q_stab) + SOFTMAX_EPS
    k_prime = jnp.exp(k_sc - k_stab) + SOFTMAX_EPS
    kstab_ref[...] = k_stab.reshape(1, 1, 1)
    pls_ref[0] = q_stab + (k_stab - 2.0 * HALF_LOG_NB)

    v_ext = jnp.concatenate([v, jnp.ones_like(qn2)], axis=-1)
    kv_ext = _bdot(k_prime, v_ext, (((0,), (0,)), ((), ())))
    qkcat = _bdot(q_prime, kv_ext)
    qk1_ref[0] = qkcat[:, E:E + 1]
    qkv_ref[0] = qkcat[:, :E]


def _prep_call(q3, k3, v3, alpha, beta, proj):
    grid = (BH,)
    row = lambda b: (b, 0, 0)
    out_shapes = (
        jax.ShapeDtypeStruct((BH, T, N_HASHES), jnp.float32),
        jax.ShapeDtypeStruct((BH, T, N_HASHES), jnp.float32),
        jax.ShapeDtypeStruct((BH, T, 1), jnp.float32),
        jax.ShapeDtypeStruct((BH, 1, 1), jnp.float32),
        jax.ShapeDtypeStruct((BH, T, 1), jnp.float32),
        jax.ShapeDtypeStruct((BH, T, E), jnp.float32),
    )
    return pl.pallas_call(
        _prep_body,
        grid=grid,
        in_specs=[
            pl.BlockSpec((1, T, E), row),
            pl.BlockSpec((1, T, E), row),
            pl.BlockSpec((1, T, E), row),
            pl.BlockSpec((E + 2, N_HASHES), lambda b: (0, 0)),
            pl.BlockSpec((1, N_HASHES), lambda b: (0, 0)),
            pl.BlockSpec((E, NB), lambda b: (0, 0)),
        ],
        out_specs=(
            pl.BlockSpec((1, T, N_HASHES), row),
            pl.BlockSpec((1, T, N_HASHES), row),
            pl.BlockSpec((1, T, 1), row),
            pl.BlockSpec((1, 1, 1), lambda b: (b, 0, 0)),
            pl.BlockSpec((1, T, 1), row),
            pl.BlockSpec((1, T, E), row),
        ),
        out_shape=out_shapes,
    )(q3, k3, v3, alpha, beta, proj)


CT = 512
CB = CT // BUCKET


def _bucket_body(sq_ref, sk_ref, sv_ref, spls_ref, sqb_ref, skbt_ref,
                 kstab_ref, proj_ref, so_ref, slse_ref, sds_ref):
    proj = proj_ref[...]
    kstab = kstab_ref[...].reshape(1, 1)
    for n in range(CB):
        s = n * BUCKET
        qb = sq_ref[0, 0, s:s + BUCKET, :]
        kb = sk_ref[0, 0, s:s + BUCKET, :]
        vb = sv_ref[0, 0, s:s + BUCKET, :]
        splsb = spls_ref[0, 0, s:s + BUCKET, :]
        sqbb = sqb_ref[0, 0, s:s + BUCKET, :]
        skbb = skbt_ref[0, 0, :, s:s + BUCKET]

        qn2 = jnp.sum(qb * qb, axis=-1, keepdims=True)
        kn2 = jnp.sum(kb * kb, axis=-1, keepdims=True)
        q_sc = _bdot(SQRT_TEMP * qb, proj) - qn2 * (SM_TEMP * 0.5)
        k_sc = _bdot(SQRT_TEMP * kb, proj) - kn2 * (SM_TEMP * 0.5)
        q_stab = jnp.max(q_sc, axis=-1, keepdims=True)
        qp = jnp.exp(q_sc - q_stab) + SOFTMAX_EPS
        kp = jnp.exp(k_sc - kstab) + SOFTMAX_EPS

        inner = _bdot(qb, kb, (((1,), (1,)), ((), ()))) * SM_TEMP
        dp = _bdot(qp, kp, (((1,), (1,)), ((), ())))
        dup = (sqbb == skbb)
        inner = inner - jnp.where(dup, math.log(2.0), 0.0)
        dp = jnp.where(dup, dp * 0.5, dp)

        lse = jnp.maximum(jnp.max(inner, axis=-1, keepdims=True), splsb)
        dots = jnp.exp(inner - lse) - dp * jnp.exp(splsb - lse)
        ob = _bdot(dots, vb)
        so_ref[0, 0, s:s + BUCKET, :] = ob
        slse_ref[0, 0, s:s + BUCKET, :] = lse
        sds_ref[0, 0, s:s + BUCKET, :] = jnp.sum(dots, axis=-1, keepdims=True)


def _bucket_call(sq, sk, sv, spls, sqb, skbt, kstab, proj):
    grid = (N_HASHES, BH, T // CT)
    blk = lambda h, b, t: (h, b, t, 0)
    out_shapes = (
        jax.ShapeDtypeStruct((N_HASHES, BH, T, E), jnp.float32),
        jax.ShapeDtypeStruct((N_HASHES, BH, T, 1), jnp.float32),
        jax.ShapeDtypeStruct((N_HASHES, BH, T, 1), jnp.float32),
    )
    return pl.pallas_call(
        _bucket_body,
        grid=grid,
        in_specs=[
            pl.BlockSpec((1, 1, CT, E), blk),
            pl.BlockSpec((1, 1, CT, E), blk),
            pl.BlockSpec((1, 1, CT, E), blk),
            pl.BlockSpec((1, 1, CT, 1), blk),
            pl.BlockSpec((1, 1, CT, 1), blk),
            pl.BlockSpec((1, 1, 1, CT), lambda h, b, t: (h, b, 0, t)),
            pl.BlockSpec((1, 1, 1), lambda h, b, t: (b, 0, 0)),
            pl.BlockSpec((E, NB), lambda h, b, t: (0, 0)),
        ],
        out_specs=(
            pl.BlockSpec((1, 1, CT, E), blk),
            pl.BlockSpec((1, 1, CT, 1), blk),
            pl.BlockSpec((1, 1, CT, 1), blk),
        ),
        out_shape=out_shapes,
    )(sq, sk, sv, spls, sqb, skbt, kstab, proj)


def _combine_body(o_ref, lse_ref, ds_ref, pls_ref, qk1_ref, qkv_ref, out_ref):
    l0 = lse_ref[0, 0]
    l1 = lse_ref[1, 0]
    m = jnp.maximum(l0, l1)
    nls = m + jnp.log(jnp.exp(l0 - m) + jnp.exp(l1 - m))
    p0 = jnp.exp(l0 - nls)
    p1 = jnp.exp(l1 - nls)
    out = o_ref[0, 0] * p0 + o_ref[1, 0] * p1
    psc = jnp.exp(pls_ref[0] - nls)
    out = out + qkv_ref[0] * psc
    norm = ds_ref[0, 0] * p0 + ds_ref[1, 0] * p1 + qk1_ref[0] * psc
    out_ref[0] = out / jnp.maximum(norm, 1e-6)


def _combine_call(o_u, lse_u, ds_u, pls, qk1, qkv):
    grid = (BH,)
    hrow = lambda b: (0, b, 0, 0)
    row = lambda b: (b, 0, 0)
    return pl.pallas_call(
        _combine_body,
        grid=grid,
        in_specs=[
            pl.BlockSpec((N_HASHES, 1, T, E), hrow),
            pl.BlockSpec((N_HASHES, 1, T, 1), hrow),
            pl.BlockSpec((N_HASHES, 1, T, 1), hrow),
            pl.BlockSpec((1, T, 1), row),
            pl.BlockSpec((1, T, 1), row),
            pl.BlockSpec((1, T, E), row),
        ],
        out_specs=pl.BlockSpec((1, T, E), row),
        out_shape=jax.ShapeDtypeStruct((BH, T, E), jnp.float32),
    )(o_u, lse_u, ds_u, pls, qk1, qkv)


OT = 1024


def _outproj_body(x_ref, w_ref, b_ref, out_ref):
    out_ref[0] = _bdot(x_ref[0], w_ref[...]) + b_ref[...]


def _outproj_call(x, w, b2):
    grid = (B, T // OT)
    return pl.pallas_call(
        _outproj_body,
        grid=grid,
        in_specs=[
            pl.BlockSpec((1, OT, H * E), lambda i, t: (i, t, 0)),
            pl.BlockSpec((H * E, E), lambda i, t: (0, 0)),
            pl.BlockSpec((1, E), lambda i, t: (0, 0)),
        ],
        out_specs=pl.BlockSpec((1, OT, E), lambda i, t: (i, t, 0)),
        out_shape=jax.ShapeDtypeStruct((B, T, E), jnp.float32),
    )(x, w, b2)


def kernel(query, key, value, alpha, beta, proj, W_out, b_out):
    q3 = query.reshape(B, T, H, E).transpose(0, 2, 1, 3).reshape(BH, T, E)
    k3 = key.reshape(B, T, H, E).transpose(0, 2, 1, 3).reshape(BH, T, E)
    v3 = value.reshape(B, T, H, E).transpose(0, 2, 1, 3).reshape(BH, T, E)

    hq, hk, pls, kstab, qk1, qkv = _prep_call(q3, k3, v3, alpha, beta, proj)
    hq = hq.transpose(2, 0, 1)
    hk = hk.transpose(2, 0, 1)

    permq = jnp.argsort(hq, axis=-1)
    permk = jnp.argsort(hk, axis=-1)
    def _inv_perm(perm):
        rows = N_HASHES * BH
        p = perm.reshape(rows, T)
        ridx = jnp.arange(rows, dtype=jnp.int32)[:, None]
        io = jnp.broadcast_to(jnp.arange(T, dtype=jnp.int32)[None, :], (rows, T))
        inv = jnp.zeros((rows, T), jnp.int32).at[ridx, p].set(
            io, unique_indices=True)
        return inv.reshape(N_HASHES, BH, T)

    rankq = _inv_perm(permq)
    rankk = _inv_perm(permk)
    qbuck = rankq // BUCKET
    kbuck = rankk // BUCKET

    def gather_rows(x, perm):
        return x[jnp.arange(BH)[None, :, None], perm]

    sq = gather_rows(q3, permq)
    sk = gather_rows(k3, permk)
    sv = gather_rows(v3, permk)
    spls = gather_rows(pls, permq)
    oq = qbuck[::-1]
    ok = kbuck[::-1]
    sqb = jnp.take_along_axis(oq, permq, axis=-1)[..., None].astype(jnp.int32)
    skbt = jnp.take_along_axis(ok, permk, axis=-1)[:, :, None, :].astype(jnp.int32)

    so, slse, sds = _bucket_call(sq, sk, sv, spls, sqb, skbt, kstab, proj)

    def unsort(x, rank):
        return jnp.take_along_axis(x, rank[..., None], axis=2)

    o_u = unsort(so, rankq)
    lse_u = unsort(slse, rankq)
    ds_u = unsort(sds, rankq)

    outn = _combine_call(o_u, lse_u, ds_u, pls, qk1, qkv)
    x = outn.reshape(B, H, T, E).transpose(0, 2, 1, 3).reshape(B, T, H * E)
    return _outproj_call(x, W_out, b_out.reshape(1, E))


# speedup vs baseline: 2.1947x; 2.1947x over previous
import math

import jax
import jax.numpy as jnp
from jax import lax
from jax.experimental import pallas as pl
from jax.experimental.pallas import tpu as pltpu

BF = jnp.bfloat16
F32 = jnp.float32


def _bdot(a, b, dims=None):
    if dims is None:
        dims = (((a.ndim - 1,), (0,)), ((), ()))
    return lax.dot_general(a.astype(BF), b.astype(BF), dims,
                           preferred_element_type=F32)


B, T, D = 2, 4096, 1024
H, E = 16, 64
BH = B * H
NB = 128
BUCKET = 64
N_HASHES = 2
SOFTMAX_EPS = 1e-06
SM_TEMP = 1.0 / math.sqrt(E)
SQRT_TEMP = math.sqrt(SM_TEMP)
HALF_LOG_NB = 0.5 * math.log(NB)


def _prep_body(q_ref, k_ref, v_ref, alpha_ref, beta_ref, proj_ref,
               hq_ref, hk_ref, pls_ref, kstab_ref, qk1_ref, qkv_ref):
    q = q_ref[0]
    k = k_ref[0]
    v = v_ref[0]
    alpha = alpha_ref[...]
    beta = beta_ref[...]
    proj = proj_ref[...]

    qn2 = jnp.sum(q * q, axis=-1, keepdims=True)
    kn2 = jnp.sum(k * k, axis=-1, keepdims=True)
    qn = jnp.sqrt(qn2)
    kn = jnp.sqrt(kn2)
    mq = jnp.max(qn)
    mk = jnp.max(kn)
    q_extra = jnp.sqrt(jnp.maximum(mq * mq - qn * qn, 0.0))
    k_extra = jnp.sqrt(jnp.maximum(mk * mk - kn * kn, 0.0))
    zcol = jnp.zeros_like(qn)
    q_ext = jnp.concatenate([q, q_extra, zcol], axis=-1).astype(BF)
    k_ext = jnp.concatenate([k, zcol, k_extra], axis=-1).astype(BF)
    alpha_bf = alpha.astype(BF)
    hq = jnp.dot(q_ext, alpha_bf, preferred_element_type=F32) + beta
    hk = jnp.dot(k_ext, alpha_bf, preferred_element_type=F32) + beta
    hq_ref[0] = hq
    hk_ref[0] = hk

    q_sc = _bdot(SQRT_TEMP * q, proj) - qn2 * (SM_TEMP * 0.5)
    k_sc = _bdot(SQRT_TEMP * k, proj) - kn2 * (SM_TEMP * 0.5)
    q_stab = jnp.max(q_sc, axis=-1, keepdims=True)
    k_stab = jnp.max(k_sc)
    q_prime = jnp.exp(q_sc - q_stab) + SOFTMAX_EPS
    k_prime = jnp.exp(k_sc - k_stab) + SOFTMAX_EPS
    kstab_ref[...] = k_stab.reshape(1, 1, 1)
    pls_ref[0] = q_stab + (k_stab - 2.0 * HALF_LOG_NB)

    v_ext = jnp.concatenate([v, jnp.ones_like(qn2)], axis=-1)
    kv_ext = _bdot(k_prime, v_ext, (((0,), (0,)), ((), ())))
    qkcat = _bdot(q_prime, kv_ext)
    qk1_ref[0] = qkcat[:, E:E + 1]
    qkv_ref[0] = qkcat[:, :E]


def _prep_call(q3, k3, v3, alpha, beta, proj):
    grid = (BH,)
    row = lambda b: (b, 0, 0)
    out_shapes = (
        jax.ShapeDtypeStruct((BH, T, N_HASHES), F32),
        jax.ShapeDtypeStruct((BH, T, N_HASHES), F32),
        jax.ShapeDtypeStruct((BH, T, 1), F32),
        jax.ShapeDtypeStruct((BH, 1, 1), F32),
        jax.ShapeDtypeStruct((BH, T, 1), F32),
        jax.ShapeDtypeStruct((BH, T, E), F32),
    )
    return pl.pallas_call(
        _prep_body,
        grid=grid,
        in_specs=[
            pl.BlockSpec((1, T, E), row),
            pl.BlockSpec((1, T, E), row),
            pl.BlockSpec((1, T, E), row),
            pl.BlockSpec((E + 2, N_HASHES), lambda b: (0, 0)),
            pl.BlockSpec((1, N_HASHES), lambda b: (0, 0)),
            pl.BlockSpec((E, NB), lambda b: (0, 0)),
        ],
        out_specs=(
            pl.BlockSpec((1, T, N_HASHES), row),
            pl.BlockSpec((1, T, N_HASHES), row),
            pl.BlockSpec((1, T, 1), row),
            pl.BlockSpec((1, 1, 1), lambda b: (b, 0, 0)),
            pl.BlockSpec((1, T, 1), row),
            pl.BlockSpec((1, T, E), row),
        ),
        out_shape=out_shapes,
    )(q3, k3, v3, alpha, beta, proj)


CT = 512
CB = CT // BUCKET


def _hilo(x):
    hi = x.astype(BF)
    lo = (x - hi.astype(F32)).astype(BF)
    return hi, lo


def _gmm(oh, hi, lo=None):
    d = (((1,), (0,)), ((), ()))
    r = lax.dot_general(oh, hi, d, preferred_element_type=F32)
    if lo is not None:
        r = r + lax.dot_general(oh, lo, d, preferred_element_type=F32)
    return r


def _bucket_body(q_ref, k_ref, v_ref, pls_ref, pq_ref, pk_ref,
                 sqb_ref, skbt_ref, kstab_ref, proj_ref,
                 so_ref, slse_ref, sds_ref):
    proj = proj_ref[...]
    kstab = kstab_ref[...].reshape(1, 1)
    iota = lax.broadcasted_iota(jnp.int32, (1, T), 1)

    qhi, qlo = _hilo(q_ref[0])
    khi, klo = _hilo(k_ref[0])
    vbf = v_ref[0].astype(BF)
    phi, plo = _hilo(pls_ref[0])

    ohq = (pq_ref[0, 0] == iota).astype(BF)
    ohk = (pk_ref[0, 0] == iota).astype(BF)

    sq = _gmm(ohq, qhi, qlo)
    sk = _gmm(ohk, khi, klo)
    sv = _gmm(ohk, vbf)
    spls = _gmm(ohq, phi, plo)

    for n in range(CB):
        s = n * BUCKET
        qb = sq[s:s + BUCKET, :]
        kb = sk[s:s + BUCKET, :]
        vb = sv[s:s + BUCKET, :]
        splsb = spls[s:s + BUCKET, :]
        sqbb = sqb_ref[0, 0, s:s + BUCKET, :]
        skbb = skbt_ref[0, 0, :, s:s + BUCKET]

        qn2 = jnp.sum(qb * qb, axis=-1, keepdims=True)
        kn2 = jnp.sum(kb * kb, axis=-1, keepdims=True)
        q_sc = _bdot(SQRT_TEMP * qb, proj) - qn2 * (SM_TEMP * 0.5)
        k_sc = _bdot(SQRT_TEMP * kb, proj) - kn2 * (SM_TEMP * 0.5)
        q_stab = jnp.max(q_sc, axis=-1, keepdims=True)
        qp = jnp.exp(q_sc - q_stab) + SOFTMAX_EPS
        kp = jnp.exp(k_sc - kstab) + SOFTMAX_EPS

        inner = _bdot(qb, kb, (((1,), (1,)), ((), ()))) * SM_TEMP
        dp = _bdot(qp, kp, (((1,), (1,)), ((), ())))
        dup = (sqbb == skbb)
        inner = inner - jnp.where(dup, math.log(2.0), 0.0)
        dp = jnp.where(dup, dp * 0.5, dp)

        lse = jnp.maximum(jnp.max(inner, axis=-1, keepdims=True), splsb)
        dots = jnp.exp(inner - lse) - dp * jnp.exp(splsb - lse)
        ob = _bdot(dots, vb)
        so_ref[0, 0, s:s + BUCKET, :] = ob
        slse_ref[0, 0, s:s + BUCKET, :] = lse
        sds_ref[0, 0, s:s + BUCKET, :] = jnp.sum(dots, axis=-1, keepdims=True)


def _bucket_call(q3, k3, v3, pls, pq4, pk4, sqb, skbt, kstab, proj):
    grid = (N_HASHES, BH, T // CT)
    blk = lambda h, b, t: (h, b, t, 0)
    brow = lambda h, b, t: (b, 0, 0)
    out_shapes = (
        jax.ShapeDtypeStruct((N_HASHES, BH, T, E), F32),
        jax.ShapeDtypeStruct((N_HASHES, BH, T, 1), F32),
        jax.ShapeDtypeStruct((N_HASHES, BH, T, 1), F32),
    )
    return pl.pallas_call(
        _bucket_body,
        grid=grid,
        in_specs=[
            pl.BlockSpec((1, T, E), brow),
            pl.BlockSpec((1, T, E), brow),
            pl.BlockSpec((1, T, E), brow),
            pl.BlockSpec((1, T, 1), brow),
            pl.BlockSpec((1, 1, CT, 1), blk),
            pl.BlockSpec((1, 1, CT, 1), blk),
            pl.BlockSpec((1, 1, CT, 1), blk),
            pl.BlockSpec((1, 1, 1, CT), lambda h, b, t: (h, b, 0, t)),
            pl.BlockSpec((1, 1, 1), lambda h, b, t: (b, 0, 0)),
            pl.BlockSpec((E, NB), lambda h, b, t: (0, 0)),
        ],
        out_specs=(
            pl.BlockSpec((1, 1, CT, E), blk),
            pl.BlockSpec((1, 1, CT, 1), blk),
            pl.BlockSpec((1, 1, CT, 1), blk),
        ),
        out_shape=out_shapes,
    )(q3, k3, v3, pls, pq4, pk4, sqb, skbt, kstab, proj)


CU = 512


def _combine_body(so_ref, slse_ref, sds_ref, rk_ref, pls_ref, qk1_ref,
                  qkv_ref, out_ref):
    iota = lax.broadcasted_iota(jnp.int32, (1, T), 1)

    def unsort(h):
        oh = (rk_ref[h, 0] == iota).astype(BF)
        shi, slo = _hilo(so_ref[h, 0])
        o = _gmm(oh, shi, slo)
        cat = jnp.concatenate([slse_ref[h, 0], sds_ref[h, 0]], axis=-1)
        chi, clo = _hilo(cat)
        lsds = _gmm(oh, chi, clo)
        return o, lsds[:, :1], lsds[:, 1:2]

    o0, l0, d0 = unsort(0)
    o1, l1, d1 = unsort(1)
    m = jnp.maximum(l0, l1)
    nls = m + jnp.log(jnp.exp(l0 - m) + jnp.exp(l1 - m))
    p0 = jnp.exp(l0 - nls)
    p1 = jnp.exp(l1 - nls)
    out = o0 * p0 + o1 * p1
    psc = jnp.exp(pls_ref[0] - nls)
    out = out + qkv_ref[0] * psc
    norm = d0 * p0 + d1 * p1 + qk1_ref[0] * psc
    out_ref[0] = out / jnp.maximum(norm, 1e-6)


def _combine_call(so, slse, sds, rk4, pls, qk1, qkv):
    grid = (BH, T // CU)
    hfull = lambda b, t: (0, b, 0, 0)
    hblk = lambda b, t: (0, b, t, 0)
    blk = lambda b, t: (b, t, 0)
    return pl.pallas_call(
        _combine_body,
        grid=grid,
        in_specs=[
            pl.BlockSpec((N_HASHES, 1, T, E), hfull),
            pl.BlockSpec((N_HASHES, 1, T, 1), hfull),
            pl.BlockSpec((N_HASHES, 1, T, 1), hfull),
            pl.BlockSpec((N_HASHES, 1, CU, 1), hblk),
            pl.BlockSpec((1, CU, 1), blk),
            pl.BlockSpec((1, CU, 1), blk),
            pl.BlockSpec((1, CU, E), blk),
        ],
        out_specs=pl.BlockSpec((1, CU, E), blk),
        out_shape=jax.ShapeDtypeStruct((BH, T, E), F32),
    )(so, slse, sds, rk4, pls, qk1, qkv)


OT = 1024


def _outproj_body(x_ref, w_ref, b_ref, out_ref):
    out_ref[0] = _bdot(x_ref[0], w_ref[...]) + b_ref[...]


def _outproj_call(x, w, b2):
    grid = (B, T // OT)
    return pl.pallas_call(
        _outproj_body,
        grid=grid,
        in_specs=[
            pl.BlockSpec((1, OT, H * E), lambda i, t: (i, t, 0)),
            pl.BlockSpec((H * E, E), lambda i, t: (0, 0)),
            pl.BlockSpec((1, E), lambda i, t: (0, 0)),
        ],
        out_specs=pl.BlockSpec((1, OT, E), lambda i, t: (i, t, 0)),
        out_shape=jax.ShapeDtypeStruct((B, T, E), F32),
    )(x, w, b2)


def kernel(query, key, value, alpha, beta, proj, W_out, b_out):
    q3 = query.reshape(B, T, H, E).transpose(0, 2, 1, 3).reshape(BH, T, E)
    k3 = key.reshape(B, T, H, E).transpose(0, 2, 1, 3).reshape(BH, T, E)
    v3 = value.reshape(B, T, H, E).transpose(0, 2, 1, 3).reshape(BH, T, E)

    hq, hk, pls, kstab, qk1, qkv = _prep_call(q3, k3, v3, alpha, beta, proj)
    hq = hq.transpose(2, 0, 1)
    hk = hk.transpose(2, 0, 1)

    permq = jnp.argsort(hq, axis=-1)
    permk = jnp.argsort(hk, axis=-1)
    rankq = jnp.argsort(permq, axis=-1)
    rankk = jnp.argsort(permk, axis=-1)
    qbuck = rankq // BUCKET
    kbuck = rankk // BUCKET

    oq = qbuck[::-1]
    ok = kbuck[::-1]
    sqb = jnp.take_along_axis(oq, permq, axis=-1)[..., None].astype(jnp.int32)
    skbt = jnp.take_along_axis(ok, permk, axis=-1)[:, :, None, :].astype(
        jnp.int32)

    pq4 = permq[..., None].astype(jnp.int32)
    pk4 = permk[..., None].astype(jnp.int32)
    so, slse, sds = _bucket_call(q3, k3, v3, pls, pq4, pk4, sqb, skbt,
                                 kstab, proj)

    rk4 = rankq[..., None].astype(jnp.int32)
    outn = _combine_call(so, slse, sds, rk4, pls, qk1, qkv)
    x = outn.reshape(B, H, T, E).transpose(0, 2, 1, 3).reshape(B, T, H * E)
    return _outproj_call(x, W_out, b_out.reshape(1, E))


# speedup vs baseline: 3.6777x; 1.6757x over previous
import math

import jax
import jax.numpy as jnp
from jax import lax
from jax.experimental import pallas as pl
from jax.experimental.pallas import tpu as pltpu

BF = jnp.bfloat16
F32 = jnp.float32


def _bdot(a, b, dims=None):
    if dims is None:
        dims = (((a.ndim - 1,), (0,)), ((), ()))
    return lax.dot_general(a.astype(BF), b.astype(BF), dims,
                           preferred_element_type=F32)


B, T, D = 2, 4096, 1024
H, E = 16, 64
BH = B * H
NB = 128
BUCKET = 64
N_HASHES = 2
SOFTMAX_EPS = 1e-06
SM_TEMP = 1.0 / math.sqrt(E)
SQRT_TEMP = math.sqrt(SM_TEMP)
HALF_LOG_NB = 0.5 * math.log(NB)

QP = E + 4
KP = 2 * E + 2
OP = E + 2


def _hilo(x):
    hi = x.astype(BF)
    lo = (x - hi.astype(F32)).astype(BF)
    return hi, lo


def _prep_body(q_ref, k_ref, v_ref, alpha_ref, beta_ref, proj_ref,
               hq_ref, hk_ref, qpk_ref, kvk_ref, caux_ref, kstab_ref,
               qkv_ref):
    q = q_ref[0]
    k = k_ref[0]
    v = v_ref[0]
    alpha = alpha_ref[...]
    beta = beta_ref[...]
    proj = proj_ref[...]

    qn2 = jnp.sum(q * q, axis=-1, keepdims=True)
    kn2 = jnp.sum(k * k, axis=-1, keepdims=True)
    qn = jnp.sqrt(qn2)
    kn = jnp.sqrt(kn2)
    mq = jnp.max(qn)
    mk = jnp.max(kn)
    q_extra = jnp.sqrt(jnp.maximum(mq * mq - qn * qn, 0.0))
    k_extra = jnp.sqrt(jnp.maximum(mk * mk - kn * kn, 0.0))
    zcol = jnp.zeros_like(qn)
    q_ext = jnp.concatenate([q, q_extra, zcol], axis=-1).astype(BF)
    k_ext = jnp.concatenate([k, zcol, k_extra], axis=-1).astype(BF)
    alpha_bf = alpha.astype(BF)
    hq = jnp.dot(q_ext, alpha_bf, preferred_element_type=F32) + beta
    hk = jnp.dot(k_ext, alpha_bf, preferred_element_type=F32) + beta
    hq_ref[0] = hq
    hk_ref[0] = hk

    q_sc = _bdot(SQRT_TEMP * q, proj) - qn2 * (SM_TEMP * 0.5)
    k_sc = _bdot(SQRT_TEMP * k, proj) - kn2 * (SM_TEMP * 0.5)
    q_stab = jnp.max(q_sc, axis=-1, keepdims=True)
    k_stab = jnp.max(k_sc)
    q_prime = jnp.exp(q_sc - q_stab) + SOFTMAX_EPS
    k_prime = jnp.exp(k_sc - k_stab) + SOFTMAX_EPS
    kstab_ref[...] = k_stab.reshape(1, 1, 1)
    pls = q_stab + (k_stab - 2.0 * HALF_LOG_NB)

    pls_hi, pls_lo = _hilo(pls)
    qn2_hi, qn2_lo = _hilo(qn2)
    kn2_hi, kn2_lo = _hilo(kn2)
    qpk_ref[0] = jnp.concatenate(
        [q.astype(BF), pls_hi, pls_lo, qn2_hi, qn2_lo], axis=-1)
    kvk_ref[0] = jnp.concatenate(
        [k.astype(BF), v.astype(BF), kn2_hi, kn2_lo], axis=-1)

    v_ext = jnp.concatenate([v, jnp.ones_like(qn2)], axis=-1)
    kv_ext = _bdot(k_prime, v_ext, (((0,), (0,)), ((), ())))
    qkcat = _bdot(q_prime, kv_ext)
    caux_ref[0] = jnp.concatenate([pls, qkcat[:, E:E + 1]], axis=-1)
    qkv_ref[0] = qkcat[:, :E]


def _prep_call(q3, k3, v3, alpha, beta, proj):
    grid = (BH,)
    row = lambda b: (b, 0, 0)
    out_shapes = (
        jax.ShapeDtypeStruct((BH, T, N_HASHES), F32),
        jax.ShapeDtypeStruct((BH, T, N_HASHES), F32),
        jax.ShapeDtypeStruct((BH, T, QP), BF),
        jax.ShapeDtypeStruct((BH, T, KP), BF),
        jax.ShapeDtypeStruct((BH, T, 2), F32),
        jax.ShapeDtypeStruct((BH, 1, 1), F32),
        jax.ShapeDtypeStruct((BH, T, E), F32),
    )
    return pl.pallas_call(
        _prep_body,
        grid=grid,
        in_specs=[
            pl.BlockSpec((1, T, E), row),
            pl.BlockSpec((1, T, E), row),
            pl.BlockSpec((1, T, E), row),
            pl.BlockSpec((E + 2, N_HASHES), lambda b: (0, 0)),
            pl.BlockSpec((1, N_HASHES), lambda b: (0, 0)),
            pl.BlockSpec((E, NB), lambda b: (0, 0)),
        ],
        out_specs=(
            pl.BlockSpec((1, T, N_HASHES), row),
            pl.BlockSpec((1, T, N_HASHES), row),
            pl.BlockSpec((1, T, QP), row),
            pl.BlockSpec((1, T, KP), row),
            pl.BlockSpec((1, T, 2), row),
            pl.BlockSpec((1, 1, 1), lambda b: (b, 0, 0)),
            pl.BlockSpec((1, T, E), row),
        ),
        out_shape=out_shapes,
    )(q3, k3, v3, alpha, beta, proj)


CT = 512
CB = CT // BUCKET


def _bucket_body(qpk_ref, kvk_ref, pq_ref, pk_ref, sqb_ref, skbt_ref,
                 kstab_ref, proj_ref, olu_ref):
    proj = proj_ref[...]
    kstab = kstab_ref[...].reshape(1, 1)
    iota = lax.broadcasted_iota(jnp.int32, (1, T), 1)

    ohq = (pq_ref[0, 0] == iota).astype(BF)
    ohk = (pk_ref[0, 0] == iota).astype(BF)

    dg = (((1,), (0,)), ((), ()))
    gq = lax.dot_general(ohq, qpk_ref[0], dg, preferred_element_type=F32)
    gk = lax.dot_general(ohk, kvk_ref[0], dg, preferred_element_type=F32)
    sq = gq[:, :E]
    spls = gq[:, E:E + 1] + gq[:, E + 1:E + 2]
    sqn2 = gq[:, E + 2:E + 3] + gq[:, E + 3:E + 4]
    sk = gk[:, :E]
    sv = gk[:, E:2 * E]
    skn2 = gk[:, 2 * E:2 * E + 1] + gk[:, 2 * E + 1:2 * E + 2]

    q_sc = _bdot(SQRT_TEMP * sq, proj) - sqn2 * (SM_TEMP * 0.5)
    k_sc = _bdot(SQRT_TEMP * sk, proj) - skn2 * (SM_TEMP * 0.5)
    q_stab = jnp.max(q_sc, axis=-1, keepdims=True)
    qp_all = jnp.exp(q_sc - q_stab) + SOFTMAX_EPS
    kp_all = jnp.exp(k_sc - kstab) + SOFTMAX_EPS

    blocks = []
    for n in range(CB):
        s = n * BUCKET
        qb = sq[s:s + BUCKET, :]
        kb = sk[s:s + BUCKET, :]
        vb = sv[s:s + BUCKET, :]
        splsb = spls[s:s + BUCKET, :]
        qpb = qp_all[s:s + BUCKET, :]
        kpb = kp_all[s:s + BUCKET, :]
        sqbb = sqb_ref[0, 0, s:s + BUCKET, :]
        skbb = skbt_ref[0, 0, :, s:s + BUCKET]

        inner = _bdot(qb, kb, (((1,), (1,)), ((), ()))) * SM_TEMP
        dp = _bdot(qpb, kpb, (((1,), (1,)), ((), ())))
        dup = (sqbb == skbb)
        inner = inner - jnp.where(dup, math.log(2.0), 0.0)
        dp = jnp.where(dup, dp * 0.5, dp)

        lse = jnp.maximum(jnp.max(inner, axis=-1, keepdims=True), splsb)
        dots = jnp.exp(inner - lse) - dp * jnp.exp(splsb - lse)
        ob = _bdot(dots, vb)
        ds = jnp.sum(dots, axis=-1, keepdims=True)
        blocks.append(jnp.concatenate([ob, lse, ds], axis=-1))

    pck = jnp.concatenate(blocks, axis=0)
    hi, lo = _hilo(pck)
    ds0 = (((0,), (0,)), ((), ()))
    contrib = (lax.dot_general(ohq, hi, ds0, preferred_element_type=F32)
               + lax.dot_general(ohq, lo, ds0, preferred_element_type=F32))

    t = pl.program_id(2)

    @pl.when(t == 0)
    def _():
        olu_ref[0, 0] = contrib

    @pl.when(t != 0)
    def _():
        olu_ref[0, 0] += contrib


def _bucket_call(qpk, kvk, pq4, pk4, sqb, skbt, kstab, proj):
    grid = (N_HASHES, BH, T // CT)
    blk = lambda h, b, t: (h, b, t, 0)
    brow = lambda h, b, t: (b, 0, 0)
    return pl.pallas_call(
        _bucket_body,
        grid=grid,
        in_specs=[
            pl.BlockSpec((1, T, QP), brow),
            pl.BlockSpec((1, T, KP), brow),
            pl.BlockSpec((1, 1, CT, 1), blk),
            pl.BlockSpec((1, 1, CT, 1), blk),
            pl.BlockSpec((1, 1, CT, 1), blk),
            pl.BlockSpec((1, 1, 1, CT), lambda h, b, t: (h, b, 0, t)),
            pl.BlockSpec((1, 1, 1), lambda h, b, t: (b, 0, 0)),
            pl.BlockSpec((E, NB), lambda h, b, t: (0, 0)),
        ],
        out_specs=pl.BlockSpec((1, 1, T, OP), lambda h, b, t: (h, b, 0, 0)),
        out_shape=jax.ShapeDtypeStruct((N_HASHES, BH, T, OP), F32),
    )(qpk, kvk, pq4, pk4, sqb, skbt, kstab, proj)


CU = 1024


def _combine_body(olu_ref, caux_ref, qkv_ref, out_ref):
    x0 = olu_ref[0, 0]
    x1 = olu_ref[1, 0]
    l0 = x0[:, E:E + 1]
    d0 = x0[:, E + 1:E + 2]
    l1 = x1[:, E:E + 1]
    d1 = x1[:, E + 1:E + 2]
    m = jnp.maximum(l0, l1)
    nls = m + jnp.log(jnp.exp(l0 - m) + jnp.exp(l1 - m))
    p0 = jnp.exp(l0 - nls)
    p1 = jnp.exp(l1 - nls)
    out = x0[:, :E] * p0 + x1[:, :E] * p1
    psc = jnp.exp(caux_ref[0][:, 0:1] - nls)
    out = out + qkv_ref[0] * psc
    norm = d0 * p0 + d1 * p1 + caux_ref[0][:, 1:2] * psc
    out_ref[0] = out / jnp.maximum(norm, 1e-6)


def _combine_call(olu, caux, qkv):
    grid = (BH, T // CU)
    hblk = lambda b, t: (0, b, t, 0)
    blk = lambda b, t: (b, t, 0)
    return pl.pallas_call(
        _combine_body,
        grid=grid,
        in_specs=[
            pl.BlockSpec((N_HASHES, 1, CU, OP), hblk),
            pl.BlockSpec((1, CU, 2), blk),
            pl.BlockSpec((1, CU, E), blk),
        ],
        out_specs=pl.BlockSpec((1, CU, E), blk),
        out_shape=jax.ShapeDtypeStruct((BH, T, E), F32),
    )(olu, caux, qkv)


OT = 1024


def _outproj_body(x_ref, w_ref, b_ref, out_ref):
    out_ref[0] = _bdot(x_ref[0], w_ref[...]) + b_ref[...]


def _outproj_call(x, w, b2):
    grid = (B, T // OT)
    return pl.pallas_call(
        _outproj_body,
        grid=grid,
        in_specs=[
            pl.BlockSpec((1, OT, H * E), lambda i, t: (i, t, 0)),
            pl.BlockSpec((H * E, E), lambda i, t: (0, 0)),
            pl.BlockSpec((1, E), lambda i, t: (0, 0)),
        ],
        out_specs=pl.BlockSpec((1, OT, E), lambda i, t: (i, t, 0)),
        out_shape=jax.ShapeDtypeStruct((B, T, E), F32),
    )(x, w, b2)


def kernel(query, key, value, alpha, beta, proj, W_out, b_out):
    q3 = query.reshape(B, T, H, E).transpose(0, 2, 1, 3).reshape(BH, T, E)
    k3 = key.reshape(B, T, H, E).transpose(0, 2, 1, 3).reshape(BH, T, E)
    v3 = value.reshape(B, T, H, E).transpose(0, 2, 1, 3).reshape(BH, T, E)

    hq, hk, qpk, kvk, caux, kstab, qkv = _prep_call(
        q3, k3, v3, alpha, beta, proj)
    hq = hq.transpose(2, 0, 1)
    hk = hk.transpose(2, 0, 1)

    permq = jnp.argsort(hq, axis=-1)
    permk = jnp.argsort(hk, axis=-1)
    rankq = jnp.argsort(permq, axis=-1)
    rankk = jnp.argsort(permk, axis=-1)
    qbuck = rankq // BUCKET
    kbuck = rankk // BUCKET

    oq = qbuck[::-1]
    ok = kbuck[::-1]
    sqb = jnp.take_along_axis(oq, permq, axis=-1)[..., None].astype(jnp.int32)
    skbt = jnp.take_along_axis(ok, permk, axis=-1)[:, :, None, :].astype(
        jnp.int32)

    pq4 = permq[..., None].astype(jnp.int32)
    pk4 = permk[..., None].astype(jnp.int32)
    olu = _bucket_call(qpk, kvk, pq4, pk4, sqb, skbt, kstab, proj)

    outn = _combine_call(olu, caux, qkv)
    x = outn.reshape(B, H, T, E).transpose(0, 2, 1, 3).reshape(B, T, H * E)
    return _outproj_call(x, W_out, b_out.reshape(1, E))
